# baseline (speedup 1.0000x reference)
import functools

import jax
import jax.numpy as jnp
from jax import lax
from jax.experimental import pallas as pl
from jax.experimental.pallas import tpu as pltpu, tpu_sc as plsc

_ROWS = 16384
_IN_COLS = 6400
_NUM_SLICES = 50
_WIDTH = 64
_OUT_COLS = _NUM_SLICES * _WIDTH

_NUM_TILES = 32
_LANES = 16
_R = 8

_SC_ROWS = 8192
_TC_ROWS = _ROWS - _SC_ROWS
_ROWS_PER_TILE = _SC_ROWS // _NUM_TILES
_CHUNKS = _ROWS_PER_TILE // _R
_TC_BLOCK_ROWS = 256


def _body(in_hbm, starts_hbm, out_hbm, sl_v, sl_s, in0, in1, ob, rs0, rs1,
          ws):
    wid = lax.axis_index("s") * 2 + lax.axis_index("c")
    r0 = wid * _ROWS_PER_TILE

    pltpu.sync_copy(starts_hbm, sl_v)
    for blk in range(4):
        vec = sl_v[pl.ds(blk * _LANES, _LANES)]
        for lane in range(_LANES):
            if blk * _LANES + lane < _NUM_SLICES:
                sl_s[blk * _LANES + lane] = vec[lane]

    inbufs = (in0, in1)
    rsems = (rs0, rs1)

    def read(c, b):
        return pltpu.make_async_copy(
            in_hbm.at[pl.ds(r0 + c * _R, _R), :], inbufs[b], rsems[b])

    def write(c):
        return pltpu.make_async_copy(
            ob, out_hbm.at[pl.ds(r0 + c * _R, _R), :], ws)

    read(0, 0).start()

    def compact(b):
        def j_body(j, carry):
            cst = pl.multiple_of(sl_s[j], _WIDTH)
            obase = pl.multiple_of(j * _WIDTH, _WIDTH)
            moves = [(r, k)
                     for r in range(_R)
                     for k in range(0, _WIDTH, _LANES)]
            depth = 8
            pending = []
            for mv in moves:
                r, k = mv
                val = inbufs[b][r, pl.ds(cst + k, _LANES)]
                pending.append((mv, val))
                if len(pending) > depth:
                    (pr, pk), pval = pending.pop(0)
                    ob[pr, pl.ds(obase + pk, _LANES)] = pval
            for (pr, pk), pval in pending:
                ob[pr, pl.ds(obase + pk, _LANES)] = pval
            return carry

        lax.fori_loop(0, _NUM_SLICES, j_body, 0)

    def chunk(co, _):
        for b in range(2):
            c = co * 2 + b

            @pl.when(c + 1 < _CHUNKS)
            def _():
                read(c + 1, 1 - b).start()

            read(c, b).wait()

            @pl.when(c >= 1)
            def _():
                write(c - 1).wait()

            compact(b)
            write(c).start()
        return 0

    lax.fori_loop(0, _CHUNKS // 2, chunk, 0)
    write(_CHUNKS - 1).wait()


def _run_sc(input_tensor, starts_padded):
    mesh = plsc.VectorSubcoreMesh(core_axis_name="c", subcore_axis_name="s")
    return pl.kernel(
        _body,
        out_type=jax.ShapeDtypeStruct((_ROWS, _OUT_COLS), jnp.float32),
        mesh=mesh,
        scratch_types=[
            pltpu.VMEM((64,), jnp.int32),
            pltpu.SMEM((64,), jnp.int32),
            pltpu.VMEM((_R, _IN_COLS), jnp.float32),
            pltpu.VMEM((_R, _IN_COLS), jnp.float32),
            pltpu.VMEM((_R, _OUT_COLS), jnp.float32),
            pltpu.SemaphoreType.DMA,
            pltpu.SemaphoreType.DMA,
            pltpu.SemaphoreType.DMA,
        ],
    )(input_tensor, starts_padded)


def _tc_body(starts_ref, in_ref, out_ref):
    for j in range(_NUM_SLICES):
        st = pl.multiple_of(starts_ref[j], 128)
        out_ref[:, j * _WIDTH:(j + 1) * _WIDTH] = in_ref[:, pl.ds(st, _WIDTH)]


def _run_tc(input_tensor, starts):
    return pl.pallas_call(
        _tc_body,
        grid_spec=pltpu.PrefetchScalarGridSpec(
            num_scalar_prefetch=1,
            grid=(_TC_ROWS // _TC_BLOCK_ROWS,),
            in_specs=[
                pl.BlockSpec(
                    (_TC_BLOCK_ROWS, _IN_COLS),
                    lambda i, starts: (i + _SC_ROWS // _TC_BLOCK_ROWS, 0)),
            ],
            out_specs=pl.BlockSpec(
                (_TC_BLOCK_ROWS, _OUT_COLS), lambda i, starts: (i, 0)),
        ),
        out_shape=jax.ShapeDtypeStruct((_TC_ROWS, _OUT_COLS), jnp.float32),
    )(starts, input_tensor)


@jax.jit
def _run(input_tensor, starts_padded, starts):
    out_sc = _run_sc(input_tensor, starts_padded)
    out_tc = _run_tc(input_tensor, starts)
    return lax.dynamic_update_slice(out_sc, out_tc, (_SC_ROWS, 0))


def kernel(input_tensor, slices):
    starts = slices[:, 0].astype(jnp.int32)
    starts_padded = jnp.pad(starts, (0, 64 - _NUM_SLICES))
    return _run(input_tensor, starts_padded, starts)


# speedup vs baseline: 1.2261x; 1.2261x over previous
import functools

import jax
import jax.numpy as jnp
from jax import lax
from jax.experimental import pallas as pl
from jax.experimental.pallas import tpu as pltpu, tpu_sc as plsc

_ROWS = 16384
_IN_COLS = 6400
_NUM_SLICES = 50
_WIDTH = 64
_OUT_COLS = _NUM_SLICES * _WIDTH

_NUM_TILES = 32
_ROWS_PER_TILE = _ROWS // _NUM_TILES
_LANES = 16
_R = 8
_CHUNKS = _ROWS_PER_TILE // _R


def _body(in_hbm, starts_hbm, out_hbm, sl_v, sl_s, in0, in1, ob, rs0, rs1,
          ws):
    wid = lax.axis_index("s") * 2 + lax.axis_index("c")
    r0 = wid * _ROWS_PER_TILE

    pltpu.sync_copy(starts_hbm, sl_v)
    for blk in range(4):
        vec = sl_v[pl.ds(blk * _LANES, _LANES)]
        for lane in range(_LANES):
            if blk * _LANES + lane < _NUM_SLICES:
                sl_s[blk * _LANES + lane] = vec[lane]

    inbufs = (in0, in1)
    rsems = (rs0, rs1)

    def read(c, b):
        return pltpu.make_async_copy(
            in_hbm.at[pl.ds(r0 + c * _R, _R), :], inbufs[b], rsems[b])

    def write(c):
        return pltpu.make_async_copy(
            ob, out_hbm.at[pl.ds(r0 + c * _R, _R), :], ws)

    read(0, 0).start()

    def compact(b):
        def j_body(j, carry):
            cst = pl.multiple_of(sl_s[j], _WIDTH)
            obase = pl.multiple_of(j * _WIDTH, _WIDTH)
            moves = [(r, k)
                     for r in range(_R)
                     for k in range(0, _WIDTH, _LANES)]
            depth = 8
            pending = []
            for mv in moves:
                r, k = mv
                val = inbufs[b][r, pl.ds(cst + k, _LANES)]
                pending.append((mv, val))
                if len(pending) > depth:
                    (pr, pk), pval = pending.pop(0)
                    ob[pr, pl.ds(obase + pk, _LANES)] = pval
            for (pr, pk), pval in pending:
                ob[pr, pl.ds(obase + pk, _LANES)] = pval
            return carry

        lax.fori_loop(0, _NUM_SLICES, j_body, 0)

    def chunk(co, _):
        for b in range(2):
            c = co * 2 + b

            @pl.when(c + 1 < _CHUNKS)
            def _():
                read(c + 1, 1 - b).start()

            read(c, b).wait()

            @pl.when(c >= 1)
            def _():
                write(c - 1).wait()

            compact(b)
            write(c).start()
        return 0

    lax.fori_loop(0, _CHUNKS // 2, chunk, 0)
    write(_CHUNKS - 1).wait()


@jax.jit
def _run(input_tensor, starts_padded):
    mesh = plsc.VectorSubcoreMesh(core_axis_name="c", subcore_axis_name="s")
    return pl.kernel(
        _body,
        out_type=jax.ShapeDtypeStruct((_ROWS, _OUT_COLS), jnp.float32),
        mesh=mesh,
        scratch_types=[
            pltpu.VMEM((64,), jnp.int32),
            pltpu.SMEM((64,), jnp.int32),
            pltpu.VMEM((_R, _IN_COLS), jnp.float32),
            pltpu.VMEM((_R, _IN_COLS), jnp.float32),
            pltpu.VMEM((_R, _OUT_COLS), jnp.float32),
            pltpu.SemaphoreType.DMA,
            pltpu.SemaphoreType.DMA,
            pltpu.SemaphoreType.DMA,
        ],
    )(input_tensor, starts_padded)


def kernel(input_tensor, slices):
    starts = slices[:, 0].astype(jnp.int32)
    starts_padded = jnp.pad(starts, (0, 64 - _NUM_SLICES))
    return _run(input_tensor, starts_padded)
